# Initial kernel scaffold; baseline (speedup 1.0000x reference)
import jax
import jax.numpy as jnp
from jax.experimental import pallas as pl


def kernel(X, Y, params, sp_A, i):
    raise NotImplementedError("write your pallas kernel here")


# speedup vs baseline: 5.0544x; 5.0544x over previous
import functools

import jax
import jax.numpy as jnp
from jax import lax
from jax.experimental import pallas as pl
from jax.experimental.pallas import tpu as pltpu
from jax.experimental.pallas import tpu_sc as plsc

_N = 10000
_E = 160000
_D = 128
_NC = 2
_NS = 16
_NW = _NC * _NS
_CBP = 128
_EPW = _E // _NW
_CH = 40
_TAIL = _EPW - (_EPW // _CBP) * _CBP
_H3 = -(-3 * _EPW // _CBP)
_HOP_RPS = 632
_HOP_ROWS = _HOP_RPS * _NS
_HIST_RPS = 1880
_HIST_ROWS = _HIST_RPS * _NS


def _sc_mesh():
    return plsc.VectorSubcoreMesh(core_axis_name="c", subcore_axis_name="s")


def _pad_idx(idx, fill):
    per = idx.shape[0] // _NW
    r = idx.reshape(_NW, per)
    padn = (-per) % _CBP
    r = jnp.concatenate([r, jnp.full((_NW, padn), fill, jnp.int32)], axis=1)
    return r.reshape(_NW, -1, _CBP)


def _sc_histogram(idx3):
    rps = _HIST_RPS

    @functools.partial(
        pl.kernel,
        out_type=jax.ShapeDtypeStruct((_NC, _HIST_ROWS, 16), jnp.float32),
        mesh=_sc_mesh(),
        compiler_params=pltpu.CompilerParams(use_tc_tiling_on_sc=False),
        scratch_types=[
            pltpu.VMEM((_H3, _CBP), jnp.int32),
            pltpu.VMEM((_CBP, 16), jnp.float32),
            pltpu.VMEM((rps, 16), jnp.float32),
            pltpu.VMEM_SHARED((_HIST_ROWS, 16), jnp.float32),
        ],
    )
    def k(idx_hbm, out_hbm, idx_v, ones_v, zstage, acc):
        cid = lax.axis_index("c")
        sid = lax.axis_index("s")
        wid = sid * _NC + cid

        @pl.loop(0, _CBP)
        def _(r):
            ones_v[r, :] = jnp.full((16,), 1.0, jnp.float32)

        @pl.loop(0, rps)
        def _(r):
            zstage[r, :] = jnp.zeros((16,), jnp.float32)

        pltpu.sync_copy(zstage, acc.at[pl.ds(sid * rps, rps)])
        pltpu.sync_copy(idx_hbm.at[wid], idx_v)
        plsc.subcore_barrier()

        @pl.loop(0, _H3)
        def _(j):
            pltpu.sync_copy(ones_v, acc.at[idx_v.at[j]], add=True)

        plsc.subcore_barrier()
        pltpu.sync_copy(acc.at[pl.ds(sid * rps, rps)],
                        out_hbm.at[cid, pl.ds(sid * rps, rps)])

    return k(idx3)


def _sc_hop(g, src3, dst3):
    rps = _HOP_RPS

    @functools.partial(
        pl.kernel,
        out_type=jax.ShapeDtypeStruct((_NC, _HOP_ROWS, 64), jnp.float32),
        mesh=_sc_mesh(),
        compiler_params=pltpu.CompilerParams(use_tc_tiling_on_sc=False),
        scratch_types=[
            pltpu.VMEM((_CH, _CBP), jnp.int32),
            pltpu.VMEM((_CH, _CBP), jnp.int32),
            pltpu.VMEM((_CBP, 64), jnp.float32),
            pltpu.VMEM((rps, 64), jnp.float32),
            pltpu.VMEM_SHARED((_HOP_ROWS, 64), jnp.float32),
            pltpu.SemaphoreType.DMA,
        ],
    )
    def k(g_hbm, src_hbm, dst_hbm, out_hbm, src_v, dst_v, rows_v, zstage,
          acc, sem):
        cid = lax.axis_index("c")
        sid = lax.axis_index("s")
        wid = sid * _NC + cid

        @pl.loop(0, rps)
        def _(r):
            @pl.loop(0, 4)
            def _(q):
                zstage[r, pl.ds(q * 16, 16)] = jnp.zeros((16,), jnp.float32)

        pltpu.sync_copy(zstage, acc.at[pl.ds(sid * rps, rps)])
        pltpu.sync_copy(src_hbm.at[wid], src_v)
        pltpu.sync_copy(dst_hbm.at[wid], dst_v)
        plsc.subcore_barrier()

        @pl.loop(0, _CH)
        def _(j):
            pltpu.async_copy(g_hbm.at[src_v.at[j]], rows_v, sem).wait()
            pltpu.sync_copy(rows_v, acc.at[dst_v.at[j]], add=True)

        plsc.subcore_barrier()
        pltpu.sync_copy(acc.at[pl.ds(sid * rps, rps)],
                        out_hbm.at[cid, pl.ds(sid * rps, rps)])

    return k(g, src3, dst3)


def _sc_edge_gather(P0, P1, i03, i13):
    @functools.partial(
        pl.kernel,
        out_type=[jax.ShapeDtypeStruct((_E, 64), jnp.float32),
                  jax.ShapeDtypeStruct((_E, 64), jnp.float32)],
        mesh=_sc_mesh(),
        compiler_params=pltpu.CompilerParams(use_tc_tiling_on_sc=False),
        scratch_types=[
            pltpu.VMEM((_CH, _CBP), jnp.int32),
            pltpu.VMEM((_CH, _CBP), jnp.int32),
            pltpu.VMEM((_CBP, 64), jnp.float32),
            pltpu.VMEM((_CBP, 64), jnp.float32),
            pltpu.SemaphoreType.DMA,
            pltpu.SemaphoreType.DMA,
        ],
    )
    def k(p0_hbm, p1_hbm, i0_hbm, i1_hbm, g0_hbm, g1_hbm, i0_v, i1_v,
          rows0, rows1, sem0, sem1):
        cid = lax.axis_index("c")
        sid = lax.axis_index("s")
        wid = sid * _NC + cid
        base = wid * _EPW

        pltpu.sync_copy(i0_hbm.at[wid], i0_v)
        pltpu.sync_copy(i1_hbm.at[wid], i1_v)
        nfull = _EPW // _CBP

        @pl.loop(0, nfull)
        def _(j):
            c0 = pltpu.async_copy(p0_hbm.at[i0_v.at[j]], rows0, sem0)
            c1 = pltpu.async_copy(p1_hbm.at[i1_v.at[j]], rows1, sem1)
            c0.wait()
            pltpu.sync_copy(rows0, g0_hbm.at[pl.ds(base + j * _CBP, _CBP)])
            c1.wait()
            pltpu.sync_copy(rows1, g1_hbm.at[pl.ds(base + j * _CBP, _CBP)])

        c0 = pltpu.async_copy(p0_hbm.at[i0_v.at[nfull]], rows0, sem0)
        c1 = pltpu.async_copy(p1_hbm.at[i1_v.at[nfull]], rows1, sem1)
        c0.wait()
        pltpu.sync_copy(rows0.at[pl.ds(0, _TAIL)],
                        g0_hbm.at[pl.ds(base + nfull * _CBP, _TAIL)])
        c1.wait()
        pltpu.sync_copy(rows1.at[pl.ds(0, _TAIL)],
                        g1_hbm.at[pl.ds(base + nfull * _CBP, _TAIL)])

    return k(P0, P1, i03, i13)


def _erf(x):
    s = jnp.sign(x)
    ax = jnp.abs(x)
    t = 1.0 / (1.0 + 0.3275911 * ax)
    poly = t * (0.254829592 + t * (-0.284496736 + t * (
        1.421413741 + t * (-1.453152027 + t * 1.061405429))))
    return s * (1.0 - poly * jnp.exp(-ax * ax))


def _gelu(x):
    return 0.5 * x * (1.0 + _erf(x * 0.7071067811865476))


def _tc(fn, out_shape, **kw):
    return pl.pallas_call(fn, out_shape=out_shape, **kw)


def _node_pre(X, dis, bn1_w, bn1_b, l1W, l1b, tag1W0):
    def f(x_ref, dis_ref, bw_ref, bb_ref, w1_ref, b1_ref, t0_ref,
          xn_ref, g0_ref, oa_ref):
        x = x_ref[...]
        mu = jnp.mean(x, axis=0, keepdims=True)
        var = jnp.mean((x - mu) ** 2, axis=0, keepdims=True)
        xn = (x - mu) / jnp.sqrt(var + 1e-5) * bw_ref[...] + bb_ref[...]
        h0 = _gelu(jnp.dot(xn, w1_ref[...].T,
                           preferred_element_type=jnp.float32) + b1_ref[...])
        xn_ref[...] = xn
        g0_ref[...] = h0 * dis_ref[...]
        oa_ref[...] = jnp.dot(h0, t0_ref[...].T,
                              preferred_element_type=jnp.float32)

    return _tc(f, [
        jax.ShapeDtypeStruct((_N, _D), jnp.float32),
        jax.ShapeDtypeStruct((_N, 64), jnp.float32),
        jax.ShapeDtypeStruct((_N, 64), jnp.float32),
    ])(X, dis, bn1_w.reshape(1, _D), bn1_b.reshape(1, _D), l1W,
       l1b.reshape(1, 64), tag1W0)


def _tag_merge(pp, oa, dis, Wk):
    def f(pp_ref, oa_ref, dis_ref, w_ref, oa2_ref, g_ref):
        p = pp_ref[0, :_N] + pp_ref[1, :_N]
        d = dis_ref[...]
        h = p * d
        oa2_ref[...] = oa_ref[...] + jnp.dot(
            h, w_ref[...].T, preferred_element_type=jnp.float32)
        g_ref[...] = h * d

    return _tc(f, [
        jax.ShapeDtypeStruct((_N, 64), jnp.float32),
        jax.ShapeDtypeStruct((_N, 64), jnp.float32),
    ])(pp, oa, dis, Wk)


def _tag1_finish(pp, oa, dis, W3, tag1b, l2W, l2b, tag2W0):
    def f(pp_ref, oa_ref, dis_ref, w3_ref, b_ref, w2_ref, b2_ref, t0_ref,
          oa2_ref, g_ref):
        p = pp_ref[0, :_N] + pp_ref[1, :_N]
        d = dis_ref[...]
        t1 = _gelu(oa_ref[...] + jnp.dot(
            p * d, w3_ref[...].T, preferred_element_type=jnp.float32)
            + b_ref[...])
        h2 = _gelu(jnp.dot(t1, w2_ref[...].T,
                           preferred_element_type=jnp.float32) + b2_ref[...])
        oa2_ref[...] = jnp.dot(h2, t0_ref[...].T,
                               preferred_element_type=jnp.float32)
        g_ref[...] = h2 * d

    return _tc(f, [
        jax.ShapeDtypeStruct((_N, 64), jnp.float32),
        jax.ShapeDtypeStruct((_N, 64), jnp.float32),
    ])(pp, oa, dis, W3, tag1b.reshape(1, 64), l2W, l2b.reshape(1, 64),
       tag2W0)


def _tag2_finish(pp, oa, dis, xn, W3, tag2b, l5W, l5b, l6W, l6b, clW, clb):
    def f(pp_ref, oa_ref, dis_ref, xn_ref, w3_ref, b_ref, w5a_ref, w5b_ref,
          b5_ref, w6_ref, b6_ref, wc_ref, bc_ref, cl_ref, ne_ref):
        p = pp_ref[0, :_N] + pp_ref[1, :_N]
        t2 = _gelu(oa_ref[...] + jnp.dot(
            p * dis_ref[...], w3_ref[...].T,
            preferred_element_type=jnp.float32) + b_ref[...])
        a1 = _gelu(jnp.dot(xn_ref[...], w5a_ref[...].T,
                           preferred_element_type=jnp.float32)
                   + jnp.dot(t2, w5b_ref[...].T,
                             preferred_element_type=jnp.float32)
                   + b5_ref[...])
        a = jnp.dot(a1, w6_ref[...].T,
                    preferred_element_type=jnp.float32) + b6_ref[...]
        cl_ref[...] = jnp.dot(_gelu(a), wc_ref[...].T,
                              preferred_element_type=jnp.float32) + bc_ref[...]
        m = jnp.max(a, axis=1, keepdims=True)
        ea = jnp.exp(a - m)
        ne_ref[...] = ea / jnp.sum(ea, axis=1, keepdims=True)

    return _tc(f, [
        jax.ShapeDtypeStruct((_N, 5), jnp.float32),
        jax.ShapeDtypeStruct((_N, 32), jnp.float32),
    ])(pp, oa, dis, xn, W3, tag2b.reshape(1, 64), l5W[:, :_D], l5W[:, _D:],
       l5b.reshape(1, 64), l6W, l6b.reshape(1, 32), clW, clb.reshape(1, 5))


def _ystats(Y):
    nblk = _E // _BE

    def f(y_ref, o_ref):
        ii = pl.program_id(0)

        @pl.when(ii == 0)
        def _():
            o_ref[...] = jnp.zeros((2, 16), jnp.float32)

        y = y_ref[...]
        s1 = jnp.sum(y, axis=0, keepdims=True)
        s2 = jnp.sum(y * y, axis=0, keepdims=True)
        o_ref[...] = o_ref[...] + jnp.concatenate([s1, s2], axis=0)

        @pl.when(ii == nblk - 1)
        def _():
            o_ref[...] = o_ref[...] * (1.0 / _E)

    return _tc(
        f, jax.ShapeDtypeStruct((2, 16), jnp.float32),
        grid=(nblk,),
        in_specs=[pl.BlockSpec((_BE, 16), lambda ii: (ii, 0))],
        out_specs=pl.BlockSpec((2, 16), lambda ii: (0, 0)),
    )(Y)


def _edge_fold(ne, X, c0, c1, ys, bn2_w, bn2_b, e1W, e1b):
    def f(ne_ref, x_ref, c0_ref, c1_ref, ys_ref, bw_ref, bb_ref, w_ref,
          b1_ref, p0_ref, p1_ref, ay_ref, cb_ref):
        ne_v = ne_ref[...]
        x = x_ref[...]
        c0 = c0_ref[...]
        c1 = c1_ref[...]
        w_all = w_ref[...]
        inv_e = 1.0 / _E
        cbar = b1_ref[...]

        def fold(tbl, cnt, off, width):
            wb = w_all[:, off:off + width]
            s1 = jnp.sum(tbl * cnt, axis=0, keepdims=True) * inv_e
            s2 = jnp.sum(tbl * tbl * cnt, axis=0, keepdims=True) * inv_e
            var = s2 - s1 * s1
            sc = lax.rsqrt(var + 1.0) * bw_ref[:, off:off + width]
            A = wb.T * sc.reshape(width, 1)
            cc = jnp.dot(bb_ref[:, off:off + width] - s1 * sc, wb.T,
                         preferred_element_type=jnp.float32)
            return A, cc

        A0, cb0 = fold(ne_v, c0, 0, 32)
        A1, cb1 = fold(ne_v, c1, 32, 32)
        A2, cb2 = fold(x, c0, 64, 32 + 96)
        A3, cb3 = fold(x, c1, 192, 128)
        muy = ys_ref[0:1, :]
        vary = ys_ref[1:2, :] - muy * muy
        scy = lax.rsqrt(vary + 1.0) * bw_ref[:, 320:336]
        ay_ref[...] = w_all[:, 320:336].T * scy.reshape(16, 1)
        cb4 = jnp.dot(bb_ref[:, 320:336] - muy * scy, w_all[:, 320:336].T,
                      preferred_element_type=jnp.float32)
        cb_ref[...] = cbar + cb0 + cb1 + cb2 + cb3 + cb4
        p0_ref[...] = (jnp.dot(ne_v, A0, preferred_element_type=jnp.float32)
                       + jnp.dot(x, A2, preferred_element_type=jnp.float32))
        p1_ref[...] = (jnp.dot(ne_v, A1, preferred_element_type=jnp.float32)
                       + jnp.dot(x, A3, preferred_element_type=jnp.float32))

    return _tc(f, [
        jax.ShapeDtypeStruct((_N, 64), jnp.float32),
        jax.ShapeDtypeStruct((_N, 64), jnp.float32),
        jax.ShapeDtypeStruct((16, 64), jnp.float32),
        jax.ShapeDtypeStruct((1, 64), jnp.float32),
    ])(ne, X, c0, c1, ys, bn2_w.reshape(1, 336), bn2_b.reshape(1, 336),
       e1W, e1b.reshape(1, 64))


_BE = 8000


def _edge_final(G0, G1, Y, Ay, cbar, e2W, e2b):
    nblk = _E // _BE

    def f(g0_ref, g1_ref, y_ref, ay_ref, cb_ref, w2_ref, b2_ref, o_ref):
        t = (g0_ref[...] + g1_ref[...]
             + jnp.dot(y_ref[...], ay_ref[...],
                       preferred_element_type=jnp.float32) + cb_ref[...])
        t = _gelu(t)
        z = jnp.sum(t * w2_ref[...], axis=1) + b2_ref[0, 0]
        o_ref[0, 0, :] = jax.nn.sigmoid(z)

    out = _tc(
        f, jax.ShapeDtypeStruct((nblk, 1, _BE), jnp.float32),
        grid=(nblk,),
        in_specs=[
            pl.BlockSpec((_BE, 64), lambda ii: (ii, 0)),
            pl.BlockSpec((_BE, 64), lambda ii: (ii, 0)),
            pl.BlockSpec((_BE, 16), lambda ii: (ii, 0)),
            pl.BlockSpec((16, 64), lambda ii: (0, 0)),
            pl.BlockSpec((1, 64), lambda ii: (0, 0)),
            pl.BlockSpec((1, 64), lambda ii: (0, 0)),
            pl.BlockSpec((1, 1), lambda ii: (0, 0)),
        ],
        out_specs=pl.BlockSpec((1, 1, _BE), lambda ii: (ii, 0, 0)),
        compiler_params=pltpu.CompilerParams(
            dimension_semantics=("parallel",)),
    )(G0, G1, Y, Ay, cbar, e2W, e2b.reshape(1, 1))
    return out.reshape(_E)


def kernel(X, Y, params, sp_A, i):
    p = params
    src, dst = sp_A[0], sp_A[1]
    i0, i1 = i[0], i[1]

    hist_idx = _pad_idx(
        jnp.concatenate([dst, i0 + _N, i1 + 2 * _N]), 3 * _N)
    src3 = _pad_idx(src, 0)
    dst3 = _pad_idx(dst, _N)
    i03 = _pad_idx(i0, 0)
    i13 = _pad_idx(i1, 0)

    cp3 = _sc_histogram(hist_idx)
    ys = _ystats(Y)

    cnt = cp3[0, :, 0] + cp3[1, :, 0]
    deg = cnt[:_N]
    dis = jnp.where(deg > 0, lax.rsqrt(deg), 0.0).reshape(_N, 1)
    c0 = cnt[_N:2 * _N].reshape(_N, 1)
    c1 = cnt[2 * _N:3 * _N].reshape(_N, 1)

    xn, g, oa = _node_pre(
        X, dis, p["bn1_w"], p["bn1_b"], p["l1W"], p["l1b"], p["tag1W"][0])

    pp = _sc_hop(g, src3, dst3)
    oa, g = _tag_merge(pp, oa, dis, p["tag1W"][1])
    pp = _sc_hop(g, src3, dst3)
    oa, g = _tag_merge(pp, oa, dis, p["tag1W"][2])
    pp = _sc_hop(g, src3, dst3)
    oa, g = _tag1_finish(pp, oa, dis, p["tag1W"][3], p["tag1b"], p["l2W"],
                         p["l2b"], p["tag2W"][0])

    pp = _sc_hop(g, src3, dst3)
    oa, g = _tag_merge(pp, oa, dis, p["tag2W"][1])
    pp = _sc_hop(g, src3, dst3)
    oa, g = _tag_merge(pp, oa, dis, p["tag2W"][2])
    pp = _sc_hop(g, src3, dst3)
    cl, ne = _tag2_finish(pp, oa, dis, xn, p["tag2W"][3], p["tag2b"],
                          p["l5W"], p["l5b"], p["l6W"], p["l6b"],
                          p["clW"], p["clb"])

    P0, P1, Ay, cbar = _edge_fold(ne, X, c0, c1, ys, p["bn2_w"], p["bn2_b"],
                                  p["e1W"], p["e1b"])
    G0, G1 = _sc_edge_gather(P0, P1, i03, i13)
    E_pred = _edge_final(G0, G1, Y, Ay, cbar, p["e2W"], p["e2b"])
    return (cl, E_pred)


# speedup vs baseline: 5.2889x; 1.0464x over previous
import functools

import jax
import jax.numpy as jnp
from jax import lax
from jax.experimental import pallas as pl
from jax.experimental.pallas import tpu as pltpu
from jax.experimental.pallas import tpu_sc as plsc

_N = 10000
_E = 160000
_D = 128
_NC = 2
_NS = 16
_NW = _NC * _NS
_CBP = 128
_EPW = _E // _NW
_CH = 40
_TAIL = _EPW - (_EPW // _CBP) * _CBP
_H3 = -(-3 * _EPW // _CBP)
_HOP_RPS = 632
_HOP_ROWS = _HOP_RPS * _NS
_HIST_RPS = 1880
_HIST_ROWS = _HIST_RPS * _NS


def _sc_mesh():
    return plsc.VectorSubcoreMesh(core_axis_name="c", subcore_axis_name="s")


def _pad_idx(idx, fill):
    per = idx.shape[0] // _NW
    r = idx.reshape(_NW, per)
    padn = (-per) % _CBP
    r = jnp.concatenate([r, jnp.full((_NW, padn), fill, jnp.int32)], axis=1)
    return r.reshape(_NW, -1, _CBP)


def _sc_histogram(idx3):
    rps = _HIST_RPS

    @functools.partial(
        pl.kernel,
        out_type=jax.ShapeDtypeStruct((_NC, _HIST_ROWS, 16), jnp.float32),
        mesh=_sc_mesh(),
        compiler_params=pltpu.CompilerParams(use_tc_tiling_on_sc=False),
        scratch_types=[
            pltpu.VMEM((_H3, _CBP), jnp.int32),
            pltpu.VMEM((_CBP, 16), jnp.float32),
            pltpu.VMEM((rps, 16), jnp.float32),
            pltpu.VMEM_SHARED((_HIST_ROWS, 16), jnp.float32),
        ],
    )
    def k(idx_hbm, out_hbm, idx_v, ones_v, zstage, acc):
        cid = lax.axis_index("c")
        sid = lax.axis_index("s")
        wid = sid * _NC + cid

        @pl.loop(0, _CBP)
        def _(r):
            ones_v[r, :] = jnp.full((16,), 1.0, jnp.float32)

        @pl.loop(0, rps)
        def _(r):
            zstage[r, :] = jnp.zeros((16,), jnp.float32)

        pltpu.sync_copy(zstage, acc.at[pl.ds(sid * rps, rps)])
        pltpu.sync_copy(idx_hbm.at[wid], idx_v)
        plsc.subcore_barrier()

        @pl.loop(0, _H3)
        def _(j):
            pltpu.sync_copy(ones_v, acc.at[idx_v.at[j]], add=True)

        plsc.subcore_barrier()
        pltpu.sync_copy(acc.at[pl.ds(sid * rps, rps)],
                        out_hbm.at[cid, pl.ds(sid * rps, rps)])

    return k(idx3)


def _sc_hop(g, src3, dst3):
    rps = _HOP_RPS

    @functools.partial(
        pl.kernel,
        out_type=jax.ShapeDtypeStruct((_NC, _HOP_ROWS, 64), jnp.float32),
        mesh=_sc_mesh(),
        compiler_params=pltpu.CompilerParams(use_tc_tiling_on_sc=False),
        scratch_types=[
            pltpu.VMEM((_CH, _CBP), jnp.int32),
            pltpu.VMEM((_CH, _CBP), jnp.int32),
            pltpu.VMEM((_CBP, 64), jnp.float32),
            pltpu.VMEM((_CBP, 64), jnp.float32),
            pltpu.VMEM((rps, 64), jnp.float32),
            pltpu.VMEM_SHARED((_HOP_ROWS, 64), jnp.float32),
            pltpu.SemaphoreType.DMA,
            pltpu.SemaphoreType.DMA,
            pltpu.SemaphoreType.DMA,
            pltpu.SemaphoreType.DMA,
        ],
    )
    def k(g_hbm, src_hbm, dst_hbm, out_hbm, src_v, dst_v, rows_a, rows_b,
          zstage, acc, sem_a, sem_b, sem_sa, sem_sb):
        cid = lax.axis_index("c")
        sid = lax.axis_index("s")
        wid = sid * _NC + cid

        @pl.loop(0, rps)
        def _(r):
            @pl.loop(0, 4)
            def _(q):
                zstage[r, pl.ds(q * 16, 16)] = jnp.zeros((16,), jnp.float32)

        pltpu.sync_copy(zstage, acc.at[pl.ds(sid * rps, rps)])
        pltpu.sync_copy(src_hbm.at[wid], src_v)
        pltpu.sync_copy(dst_hbm.at[wid], dst_v)
        plsc.subcore_barrier()

        @pl.loop(0, _CH // 2)
        def _(t):
            j = t * 2
            ga = pltpu.async_copy(g_hbm.at[src_v.at[j]], rows_a, sem_a)
            gb = pltpu.async_copy(g_hbm.at[src_v.at[j + 1]], rows_b, sem_b)
            ga.wait()
            sa = pltpu.async_copy(rows_a, acc.at[dst_v.at[j]], sem_sa,
                                  add=True)
            gb.wait()
            sb = pltpu.async_copy(rows_b, acc.at[dst_v.at[j + 1]], sem_sb,
                                  add=True)
            sa.wait()
            sb.wait()

        plsc.subcore_barrier()
        pltpu.sync_copy(acc.at[pl.ds(sid * rps, rps)],
                        out_hbm.at[cid, pl.ds(sid * rps, rps)])

    return k(g, src3, dst3)


def _sc_edge_gather(P0, P1, i03, i13):
    @functools.partial(
        pl.kernel,
        out_type=[jax.ShapeDtypeStruct((_E, 64), jnp.float32),
                  jax.ShapeDtypeStruct((_E, 64), jnp.float32)],
        mesh=_sc_mesh(),
        compiler_params=pltpu.CompilerParams(use_tc_tiling_on_sc=False),
        scratch_types=[
            pltpu.VMEM((_CH, _CBP), jnp.int32),
            pltpu.VMEM((_CH, _CBP), jnp.int32),
            pltpu.VMEM((_CBP, 64), jnp.float32),
            pltpu.VMEM((_CBP, 64), jnp.float32),
            pltpu.VMEM((_CBP, 64), jnp.float32),
            pltpu.VMEM((_CBP, 64), jnp.float32),
            pltpu.SemaphoreType.DMA,
            pltpu.SemaphoreType.DMA,
            pltpu.SemaphoreType.DMA,
            pltpu.SemaphoreType.DMA,
            pltpu.SemaphoreType.DMA,
            pltpu.SemaphoreType.DMA,
        ],
    )
    def k(p0_hbm, p1_hbm, i0_hbm, i1_hbm, g0_hbm, g1_hbm, i0_v, i1_v,
          r0a, r1a, r0b, r1b, s0a, s1a, s0b, s1b, sw0, sw1):
        cid = lax.axis_index("c")
        sid = lax.axis_index("s")
        wid = sid * _NC + cid
        base = wid * _EPW

        pltpu.sync_copy(i0_hbm.at[wid], i0_v)
        pltpu.sync_copy(i1_hbm.at[wid], i1_v)
        nfull = _EPW // _CBP

        @pl.loop(0, nfull // 2)
        def _(t):
            j = t * 2
            ga0 = pltpu.async_copy(p0_hbm.at[i0_v.at[j]], r0a, s0a)
            ga1 = pltpu.async_copy(p1_hbm.at[i1_v.at[j]], r1a, s1a)
            gb0 = pltpu.async_copy(p0_hbm.at[i0_v.at[j + 1]], r0b, s0b)
            gb1 = pltpu.async_copy(p1_hbm.at[i1_v.at[j + 1]], r1b, s1b)
            ga0.wait()
            w0 = pltpu.async_copy(
                r0a, g0_hbm.at[pl.ds(base + j * _CBP, _CBP)], sw0)
            ga1.wait()
            w1 = pltpu.async_copy(
                r1a, g1_hbm.at[pl.ds(base + j * _CBP, _CBP)], sw1)
            gb0.wait()
            w0.wait()
            w0 = pltpu.async_copy(
                r0b, g0_hbm.at[pl.ds(base + (j + 1) * _CBP, _CBP)], sw0)
            gb1.wait()
            w1.wait()
            w1 = pltpu.async_copy(
                r1b, g1_hbm.at[pl.ds(base + (j + 1) * _CBP, _CBP)], sw1)
            w0.wait()
            w1.wait()

        j = nfull - 1
        ga0 = pltpu.async_copy(p0_hbm.at[i0_v.at[j]], r0a, s0a)
        ga1 = pltpu.async_copy(p1_hbm.at[i1_v.at[j]], r1a, s1a)
        gb0 = pltpu.async_copy(p0_hbm.at[i0_v.at[nfull]], r0b, s0b)
        gb1 = pltpu.async_copy(p1_hbm.at[i1_v.at[nfull]], r1b, s1b)
        ga0.wait()
        pltpu.sync_copy(r0a, g0_hbm.at[pl.ds(base + j * _CBP, _CBP)])
        ga1.wait()
        pltpu.sync_copy(r1a, g1_hbm.at[pl.ds(base + j * _CBP, _CBP)])
        gb0.wait()
        pltpu.sync_copy(r0b.at[pl.ds(0, _TAIL)],
                        g0_hbm.at[pl.ds(base + nfull * _CBP, _TAIL)])
        gb1.wait()
        pltpu.sync_copy(r1b.at[pl.ds(0, _TAIL)],
                        g1_hbm.at[pl.ds(base + nfull * _CBP, _TAIL)])

    return k(P0, P1, i03, i13)


def _erf(x):
    s = jnp.sign(x)
    ax = jnp.abs(x)
    t = 1.0 / (1.0 + 0.3275911 * ax)
    poly = t * (0.254829592 + t * (-0.284496736 + t * (
        1.421413741 + t * (-1.453152027 + t * 1.061405429))))
    return s * (1.0 - poly * jnp.exp(-ax * ax))


def _gelu(x):
    return 0.5 * x * (1.0 + _erf(x * 0.7071067811865476))


def _tc(fn, out_shape, **kw):
    return pl.pallas_call(fn, out_shape=out_shape, **kw)


def _node_pre(X, dis, bn1_w, bn1_b, l1W, l1b, tag1W0):
    def f(x_ref, dis_ref, bw_ref, bb_ref, w1_ref, b1_ref, t0_ref,
          xn_ref, g0_ref, oa_ref):
        x = x_ref[...]
        mu = jnp.mean(x, axis=0, keepdims=True)
        var = jnp.mean((x - mu) ** 2, axis=0, keepdims=True)
        xn = (x - mu) / jnp.sqrt(var + 1e-5) * bw_ref[...] + bb_ref[...]
        h0 = _gelu(jnp.dot(xn, w1_ref[...].T,
                           preferred_element_type=jnp.float32) + b1_ref[...])
        xn_ref[...] = xn
        g0_ref[0:_N] = h0 * dis_ref[...]
        g0_ref[_N:_HOP_ROWS] = jnp.zeros((_HOP_ROWS - _N, 64), jnp.float32)
        oa_ref[...] = jnp.dot(h0, t0_ref[...].T,
                              preferred_element_type=jnp.float32)

    return _tc(f, [
        jax.ShapeDtypeStruct((_N, _D), jnp.float32),
        jax.ShapeDtypeStruct((_HOP_ROWS, 64), jnp.float32),
        jax.ShapeDtypeStruct((_N, 64), jnp.float32),
    ])(X, dis, bn1_w.reshape(1, _D), bn1_b.reshape(1, _D), l1W,
       l1b.reshape(1, 64), tag1W0)


def _tag_merge(pp, oa, dis, Wk):
    def f(pp_ref, oa_ref, dis_ref, w_ref, oa2_ref, g_ref):
        p = pp_ref[0, :_N] + pp_ref[1, :_N]
        d = dis_ref[...]
        h = p * d
        oa2_ref[...] = oa_ref[...] + jnp.dot(
            h, w_ref[...].T, preferred_element_type=jnp.float32)
        g_ref[0:_N] = h * d
        g_ref[_N:_HOP_ROWS] = jnp.zeros((_HOP_ROWS - _N, 64), jnp.float32)

    return _tc(f, [
        jax.ShapeDtypeStruct((_N, 64), jnp.float32),
        jax.ShapeDtypeStruct((_HOP_ROWS, 64), jnp.float32),
    ])(pp, oa, dis, Wk)


def _tag1_finish(pp, oa, dis, W3, tag1b, l2W, l2b, tag2W0):
    def f(pp_ref, oa_ref, dis_ref, w3_ref, b_ref, w2_ref, b2_ref, t0_ref,
          oa2_ref, g_ref):
        p = pp_ref[0, :_N] + pp_ref[1, :_N]
        d = dis_ref[...]
        t1 = _gelu(oa_ref[...] + jnp.dot(
            p * d, w3_ref[...].T, preferred_element_type=jnp.float32)
            + b_ref[...])
        h2 = _gelu(jnp.dot(t1, w2_ref[...].T,
                           preferred_element_type=jnp.float32) + b2_ref[...])
        oa2_ref[...] = jnp.dot(h2, t0_ref[...].T,
                               preferred_element_type=jnp.float32)
        g_ref[0:_N] = h2 * d
        g_ref[_N:_HOP_ROWS] = jnp.zeros((_HOP_ROWS - _N, 64), jnp.float32)

    return _tc(f, [
        jax.ShapeDtypeStruct((_N, 64), jnp.float32),
        jax.ShapeDtypeStruct((_HOP_ROWS, 64), jnp.float32),
    ])(pp, oa, dis, W3, tag1b.reshape(1, 64), l2W, l2b.reshape(1, 64),
       tag2W0)


def _tag2_finish(pp, oa, dis, xn, W3, tag2b, l5W, l5b, l6W, l6b, clW, clb):
    def f(pp_ref, oa_ref, dis_ref, xn_ref, w3_ref, b_ref, w5a_ref, w5b_ref,
          b5_ref, w6_ref, b6_ref, wc_ref, bc_ref, cl_ref, ne_ref):
        p = pp_ref[0, :_N] + pp_ref[1, :_N]
        t2 = _gelu(oa_ref[...] + jnp.dot(
            p * dis_ref[...], w3_ref[...].T,
            preferred_element_type=jnp.float32) + b_ref[...])
        a1 = _gelu(jnp.dot(xn_ref[...], w5a_ref[...].T,
                           preferred_element_type=jnp.float32)
                   + jnp.dot(t2, w5b_ref[...].T,
                             preferred_element_type=jnp.float32)
                   + b5_ref[...])
        a = jnp.dot(a1, w6_ref[...].T,
                    preferred_element_type=jnp.float32) + b6_ref[...]
        cl_ref[...] = jnp.dot(_gelu(a), wc_ref[...].T,
                              preferred_element_type=jnp.float32) + bc_ref[...]
        m = jnp.max(a, axis=1, keepdims=True)
        ea = jnp.exp(a - m)
        ne_ref[...] = ea / jnp.sum(ea, axis=1, keepdims=True)

    return _tc(f, [
        jax.ShapeDtypeStruct((_N, 5), jnp.float32),
        jax.ShapeDtypeStruct((_N, 32), jnp.float32),
    ])(pp, oa, dis, xn, W3, tag2b.reshape(1, 64), l5W[:, :_D], l5W[:, _D:],
       l5b.reshape(1, 64), l6W, l6b.reshape(1, 32), clW, clb.reshape(1, 5))


def _ystats(Y):
    nblk = _E // _BE

    def f(y_ref, o_ref):
        ii = pl.program_id(0)

        @pl.when(ii == 0)
        def _():
            o_ref[...] = jnp.zeros((2, 16), jnp.float32)

        y = y_ref[...]
        s1 = jnp.sum(y, axis=0, keepdims=True)
        s2 = jnp.sum(y * y, axis=0, keepdims=True)
        o_ref[...] = o_ref[...] + jnp.concatenate([s1, s2], axis=0)

        @pl.when(ii == nblk - 1)
        def _():
            o_ref[...] = o_ref[...] * (1.0 / _E)

    return _tc(
        f, jax.ShapeDtypeStruct((2, 16), jnp.float32),
        grid=(nblk,),
        in_specs=[pl.BlockSpec((_BE, 16), lambda ii: (ii, 0))],
        out_specs=pl.BlockSpec((2, 16), lambda ii: (0, 0)),
    )(Y)


def _edge_fold(ne, X, c0, c1, ys, bn2_w, bn2_b, e1W, e1b):
    def f(ne_ref, x_ref, c0_ref, c1_ref, ys_ref, bw_ref, bb_ref, w_ref,
          b1_ref, p0_ref, p1_ref, ay_ref, cb_ref):
        ne_v = ne_ref[...]
        x = x_ref[...]
        c0 = c0_ref[...]
        c1 = c1_ref[...]
        w_all = w_ref[...]
        inv_e = 1.0 / _E
        cbar = b1_ref[...]

        def fold(tbl, cnt, off, width):
            wb = w_all[:, off:off + width]
            s1 = jnp.sum(tbl * cnt, axis=0, keepdims=True) * inv_e
            s2 = jnp.sum(tbl * tbl * cnt, axis=0, keepdims=True) * inv_e
            var = s2 - s1 * s1
            sc = lax.rsqrt(var + 1.0) * bw_ref[:, off:off + width]
            A = wb.T * sc.reshape(width, 1)
            cc = jnp.dot(bb_ref[:, off:off + width] - s1 * sc, wb.T,
                         preferred_element_type=jnp.float32)
            return A, cc

        A0, cb0 = fold(ne_v, c0, 0, 32)
        A1, cb1 = fold(ne_v, c1, 32, 32)
        A2, cb2 = fold(x, c0, 64, 32 + 96)
        A3, cb3 = fold(x, c1, 192, 128)
        muy = ys_ref[0:1, :]
        vary = ys_ref[1:2, :] - muy * muy
        scy = lax.rsqrt(vary + 1.0) * bw_ref[:, 320:336]
        ay_ref[...] = w_all[:, 320:336].T * scy.reshape(16, 1)
        cb4 = jnp.dot(bb_ref[:, 320:336] - muy * scy, w_all[:, 320:336].T,
                      preferred_element_type=jnp.float32)
        cb_ref[...] = cbar + cb0 + cb1 + cb2 + cb3 + cb4
        p0_ref[...] = (jnp.dot(ne_v, A0, preferred_element_type=jnp.float32)
                       + jnp.dot(x, A2, preferred_element_type=jnp.float32))
        p1_ref[...] = (jnp.dot(ne_v, A1, preferred_element_type=jnp.float32)
                       + jnp.dot(x, A3, preferred_element_type=jnp.float32))

    return _tc(f, [
        jax.ShapeDtypeStruct((_N, 64), jnp.float32),
        jax.ShapeDtypeStruct((_N, 64), jnp.float32),
        jax.ShapeDtypeStruct((16, 64), jnp.float32),
        jax.ShapeDtypeStruct((1, 64), jnp.float32),
    ])(ne, X, c0, c1, ys, bn2_w.reshape(1, 336), bn2_b.reshape(1, 336),
       e1W, e1b.reshape(1, 64))


_BE = 8000


def _edge_final(G0, G1, Y, Ay, cbar, e2W, e2b):
    nblk = _E // _BE

    def f(g0_ref, g1_ref, y_ref, ay_ref, cb_ref, w2_ref, b2_ref, o_ref):
        t = (g0_ref[...] + g1_ref[...]
             + jnp.dot(y_ref[...], ay_ref[...],
                       preferred_element_type=jnp.float32) + cb_ref[...])
        t = _gelu(t)
        z = jnp.sum(t * w2_ref[...], axis=1) + b2_ref[0, 0]
        o_ref[0, 0, :] = jax.nn.sigmoid(z)

    out = _tc(
        f, jax.ShapeDtypeStruct((nblk, 1, _BE), jnp.float32),
        grid=(nblk,),
        in_specs=[
            pl.BlockSpec((_BE, 64), lambda ii: (ii, 0)),
            pl.BlockSpec((_BE, 64), lambda ii: (ii, 0)),
            pl.BlockSpec((_BE, 16), lambda ii: (ii, 0)),
            pl.BlockSpec((16, 64), lambda ii: (0, 0)),
            pl.BlockSpec((1, 64), lambda ii: (0, 0)),
            pl.BlockSpec((1, 64), lambda ii: (0, 0)),
            pl.BlockSpec((1, 1), lambda ii: (0, 0)),
        ],
        out_specs=pl.BlockSpec((1, 1, _BE), lambda ii: (ii, 0, 0)),
        compiler_params=pltpu.CompilerParams(
            dimension_semantics=("parallel",)),
    )(G0, G1, Y, Ay, cbar, e2W, e2b.reshape(1, 1))
    return out.reshape(_E)


def kernel(X, Y, params, sp_A, i):
    p = params
    src, dst = sp_A[0], sp_A[1]
    i0, i1 = i[0], i[1]

    hist_idx = _pad_idx(
        jnp.concatenate([dst, i0 + _N, i1 + 2 * _N]), 3 * _N)
    src3 = _pad_idx(src, 0)
    dst3 = _pad_idx(dst, _N)
    i03 = _pad_idx(i0, 0)
    i13 = _pad_idx(i1, 0)

    cp3 = _sc_histogram(hist_idx)
    ys = _ystats(Y)

    cnt = cp3[0, :, 0] + cp3[1, :, 0]
    deg = cnt[:_N]
    dis = jnp.where(deg > 0, lax.rsqrt(deg), 0.0).reshape(_N, 1)
    c0 = cnt[_N:2 * _N].reshape(_N, 1)
    c1 = cnt[2 * _N:3 * _N].reshape(_N, 1)

    xn, g, oa = _node_pre(
        X, dis, p["bn1_w"], p["bn1_b"], p["l1W"], p["l1b"], p["tag1W"][0])

    pp = _sc_hop(g, src3, dst3)
    oa, g = _tag_merge(pp, oa, dis, p["tag1W"][1])
    pp = _sc_hop(g, src3, dst3)
    oa, g = _tag_merge(pp, oa, dis, p["tag1W"][2])
    pp = _sc_hop(g, src3, dst3)
    oa, g = _tag1_finish(pp, oa, dis, p["tag1W"][3], p["tag1b"], p["l2W"],
                         p["l2b"], p["tag2W"][0])

    pp = _sc_hop(g, src3, dst3)
    oa, g = _tag_merge(pp, oa, dis, p["tag2W"][1])
    pp = _sc_hop(g, src3, dst3)
    oa, g = _tag_merge(pp, oa, dis, p["tag2W"][2])
    pp = _sc_hop(g, src3, dst3)
    cl, ne = _tag2_finish(pp, oa, dis, xn, p["tag2W"][3], p["tag2b"],
                          p["l5W"], p["l5b"], p["l6W"], p["l6b"],
                          p["clW"], p["clb"])

    P0, P1, Ay, cbar = _edge_fold(ne, X, c0, c1, ys, p["bn2_w"], p["bn2_b"],
                                  p["e1W"], p["e1b"])
    G0, G1 = _sc_edge_gather(P0, P1, i03, i13)
    E_pred = _edge_final(G0, G1, Y, Ay, cbar, p["e2W"], p["e2b"])
    return (cl, E_pred)
